# baseline (speedup 1.0000x reference)
import functools

import jax
import jax.numpy as jnp
from jax import lax
from jax.experimental import pallas as pl
from jax.experimental.pallas import tpu as pltpu
from jax.experimental.pallas import tpu_sc as plsc

DEPTH = 24
VQT_NUM = 10
EMBED_DIM = 1024
BATCH = 256

_info = plsc.get_sparse_core_info()
_NC = _info.num_cores
_NS = _info.num_subcores
_NL = _info.num_lanes
_NW = _NC * _NS
_R0 = 0
_R1 = 16
_RMAX = max(_R0, _R1)
assert (_R0 + _R1) * _NS == BATCH

_mesh = plsc.VectorSubcoreMesh(core_axis_name="c", subcore_axis_name="s")


@functools.partial(
    pl.kernel,
    mesh=_mesh,
    out_type=jax.ShapeDtypeStruct((VQT_NUM, BATCH, EMBED_DIM), jnp.float32),
    scratch_types=[
        pltpu.VMEM((_NL,), jnp.int32),
        pltpu.VMEM((VQT_NUM, 1, EMBED_DIM), jnp.float32),
        pltpu.SemaphoreType.DMA,
        pltpu.SemaphoreType.DMA,
    ],
)
def _vqt_expand(table_hbm, idx_hbm, out_hbm, idx_v, row_v, gsem, wsem):
    core = lax.axis_index("c")
    sub = lax.axis_index("s")
    n_mine = jnp.where(core == 0, _R0, _R1)
    base = jnp.where(core == 0, sub * _R0, _R0 * _NS + sub * _R1)
    pltpu.sync_copy(idx_hbm, idx_v)
    layer = idx_v[...][0]
    pltpu.async_copy(
        table_hbm.at[:, pl.ds(layer, 1), :], row_v, gsem
    ).wait()
    for j in range(_RMAX):
        @pl.when(j < n_mine)
        def _fire(j=j):
            pltpu.async_copy(
                row_v, out_hbm.at[:, pl.ds(base + j, 1), :], wsem
            )

    for j in range(_RMAX):
        @pl.when(j < n_mine)
        def _drain(j=j):
            pltpu.make_async_copy(
                row_v, out_hbm.at[:, pl.ds(base + j, 1), :], wsem
            ).wait()


def kernel(query_prompt_embeddings, index, batch_size):
    del batch_size
    table_t = jnp.transpose(query_prompt_embeddings, (1, 0, 2))
    idx = jnp.zeros((_NL,), jnp.int32).at[0].set(index)
    out = _vqt_expand(table_t, idx)
    return jnp.transpose(out, (1, 0, 2))


# speedup vs baseline: 1.0117x; 1.0117x over previous
import functools

import jax
import jax.numpy as jnp
from jax import lax
from jax.experimental import pallas as pl
from jax.experimental.pallas import tpu as pltpu
from jax.experimental.pallas import tpu_sc as plsc

DEPTH = 24
VQT_NUM = 10
EMBED_DIM = 1024
BATCH = 256

_info = plsc.get_sparse_core_info()
_NC = _info.num_cores
_NS = _info.num_subcores
_NL = _info.num_lanes
_NW = _NC * _NS
_R0 = 2
_R1 = 14
_RMAX = max(_R0, _R1)
assert (_R0 + _R1) * _NS == BATCH

_mesh = plsc.VectorSubcoreMesh(core_axis_name="c", subcore_axis_name="s")


@functools.partial(
    pl.kernel,
    mesh=_mesh,
    out_type=jax.ShapeDtypeStruct((VQT_NUM, BATCH, EMBED_DIM), jnp.float32),
    scratch_types=[
        pltpu.VMEM((_NL,), jnp.int32),
        pltpu.VMEM((VQT_NUM, 1, EMBED_DIM), jnp.float32),
        pltpu.SemaphoreType.DMA,
        pltpu.SemaphoreType.DMA,
    ],
)
def _vqt_expand(table_hbm, idx_hbm, out_hbm, idx_v, row_v, gsem, wsem):
    core = lax.axis_index("c")
    sub = lax.axis_index("s")
    n_mine = jnp.where(core == 0, _R0, _R1)
    base = jnp.where(core == 0, sub * _R0, _R0 * _NS + sub * _R1)
    pltpu.sync_copy(idx_hbm, idx_v)
    layer = idx_v[...][0]
    pltpu.async_copy(
        table_hbm.at[:, pl.ds(layer, 1), :], row_v, gsem
    ).wait()
    for j in range(_RMAX):
        @pl.when(j < n_mine)
        def _fire(j=j):
            pltpu.async_copy(
                row_v, out_hbm.at[:, pl.ds(base + j, 1), :], wsem
            )

    for j in range(_RMAX):
        @pl.when(j < n_mine)
        def _drain(j=j):
            pltpu.make_async_copy(
                row_v, out_hbm.at[:, pl.ds(base + j, 1), :], wsem
            ).wait()


def kernel(query_prompt_embeddings, index, batch_size):
    del batch_size
    table_t = jnp.transpose(query_prompt_embeddings, (1, 0, 2))
    idx = jnp.full((_NL,), index, jnp.int32)
    out = _vqt_expand(table_t, idx)
    return jnp.transpose(out, (1, 0, 2))
